# baseline (speedup 1.0000x reference)
import functools

import jax
import jax.numpy as jnp
from jax import lax
from jax.experimental import pallas as pl
from jax.experimental.pallas import tpu as pltpu
from jax.experimental.pallas import tpu_sc as plsc

N_CORES = 2
N_SUBCORES = 16
N_TILES = N_CORES * N_SUBCORES
CHUNK = 128
GRAN = 128
PBUF = 2
WBODY = 8
EPS = 1e-5


def _sc_mesh():
    return plsc.VectorSubcoreMesh(core_axis_name="c", subcore_axis_name="s")


def _deg_call(npad, nchunk):
    rps = npad // N_SUBCORES

    @functools.partial(
        pl.kernel,
        out_type=jax.ShapeDtypeStruct((N_CORES, npad, GRAN), jnp.float32),
        mesh=_sc_mesh(),
        scratch_types=[
            pltpu.VMEM((nchunk, CHUNK), jnp.int32),
            pltpu.VMEM((CHUNK, GRAN), jnp.float32),
            pltpu.VMEM_SHARED((npad, GRAN), jnp.float32),
            pltpu.SemaphoreType.DMA,
        ],
    )
    def deg_kernel(dst_hbm, ones_hbm, zeros_hbm, out_hbm, idx_v, ones_v, acc,
                   sem):
        c = lax.axis_index("c")
        s = lax.axis_index("s")
        wid = c * N_SUBCORES + s
        pltpu.sync_copy(zeros_hbm.at[pl.ds(s * rps, rps)],
                        acc.at[pl.ds(s * rps, rps)])
        pltpu.sync_copy(ones_hbm, ones_v)
        pltpu.sync_copy(dst_hbm.at[wid], idx_v)
        plsc.subcore_barrier()

        @pl.loop(0, nchunk)
        def _(j):
            pltpu.async_copy(ones_v, acc.at[idx_v.at[j]], sem, add=True)

        @pl.loop(0, nchunk)
        def _(j):
            pltpu.make_async_copy(ones_hbm, ones_v, sem).wait()

        plsc.subcore_barrier()
        pltpu.sync_copy(acc.at[pl.ds(s * rps, rps)],
                        out_hbm.at[c, pl.ds(s * rps, rps)])

    return deg_kernel


def _msg_call(n, d, npad, nchunk):
    rps = npad // N_SUBCORES

    @functools.partial(
        pl.kernel,
        out_type=jax.ShapeDtypeStruct((N_CORES, npad, d), jnp.float32),
        mesh=_sc_mesh(),
        scratch_types=[
            pltpu.VMEM((nchunk, CHUNK), jnp.int32),
            pltpu.VMEM((WBODY, CHUNK), jnp.int32),
            pltpu.VMEM((PBUF, CHUNK, d), jnp.float32),
            pltpu.VMEM_SHARED((npad, d), jnp.float32),
            pltpu.SemaphoreType.DMA,
            pltpu.SemaphoreType.DMA,
            pltpu.SemaphoreType.DMA,
        ],
    )
    def msg_kernel(hp_hbm, src_hbm, dst_hbm, zeros_hbm, out_hbm,
                   src_v, dwin, bufs, acc, gsem, ssem, dsem):
        c = lax.axis_index("c")
        s = lax.axis_index("s")
        wid = c * N_SUBCORES + s
        pltpu.sync_copy(zeros_hbm.at[pl.ds(s * rps, rps)],
                        acc.at[pl.ds(s * rps, rps)])
        pltpu.sync_copy(src_hbm.at[wid], src_v)
        plsc.subcore_barrier()

        nfull = (nchunk // WBODY) * WBODY

        @pl.loop(0, nfull, step=WBODY)
        def _(j0):
            dw = pltpu.async_copy(
                dst_hbm.at[wid, pl.ds(j0, WBODY)], dwin, dsem)
            gs = [pltpu.async_copy(hp_hbm.at[src_v.at[j0]], bufs.at[0],
                                   gsem)]
            ss = []
            for b in range(WBODY):
                gs[b].wait()
                if b >= 1:
                    ss[b - 1].wait()
                if b + 1 < WBODY:
                    gs.append(pltpu.async_copy(
                        hp_hbm.at[src_v.at[j0 + b + 1]],
                        bufs.at[(b + 1) % PBUF], gsem))
                if b == 0:
                    dw.wait()
                ss.append(pltpu.async_copy(
                    bufs.at[b % PBUF], acc.at[dwin.at[b]], ssem, add=True))
            ss[WBODY - 1].wait()

        ntail = nchunk - nfull
        if ntail:
            pltpu.sync_copy(dst_hbm.at[wid, pl.ds(nfull, ntail)],
                            dwin.at[pl.ds(0, ntail)])
        for j in range(nfull, nchunk):
            b = j - nfull
            pltpu.async_copy(hp_hbm.at[src_v.at[j]],
                             bufs.at[b % PBUF], gsem).wait()
            pltpu.async_copy(bufs.at[b % PBUF], acc.at[dwin.at[b]],
                             ssem, add=True).wait()

        plsc.subcore_barrier()
        pltpu.sync_copy(acc.at[pl.ds(s * rps, rps)],
                        out_hbm.at[c, pl.ds(s * rps, rps)])

    return msg_kernel


def _matmul_call(n, d):
    def body(x_ref, w_ref, o_ref):
        o_ref[...] = lax.dot_general(
            x_ref[...], w_ref[...], (((1,), (1,)), ((), ())),
            preferred_element_type=jnp.float32)

    return pl.pallas_call(
        body, out_shape=jax.ShapeDtypeStruct((n, d), jnp.float32))


def _scale_call(n, d, npad):
    def body(dp_ref, h_ref, o_ref):
        deg = dp_ref[0, :n, 0:1] + dp_ref[1, :n, 0:1] + 1.0
        o_ref[...] = h_ref[...] * lax.rsqrt(deg)

    return pl.pallas_call(
        body, out_shape=jax.ShapeDtypeStruct((n, d), jnp.float32))


def _epilogue_call(n, d, npad):
    def body(mp_ref, dp_ref, hp_ref, x_ref, b_ref, g_ref, be_ref, o_ref):
        deg = dp_ref[0, :n, 0:1] + dp_ref[1, :n, 0:1] + 1.0
        dis = lax.rsqrt(deg)
        s = (mp_ref[0, :n, :] + mp_ref[1, :n, :] + hp_ref[...]) * dis
        s = s + b_ref[...]
        mean = jnp.mean(s, axis=0, keepdims=True)
        var = jnp.mean((s - mean) ** 2, axis=0, keepdims=True)
        y = (s - mean) * lax.rsqrt(var + EPS) * g_ref[...] + be_ref[...]
        o_ref[...] = jnp.maximum(y, 0.0) + x_ref[...]

    return pl.pallas_call(
        body, out_shape=jax.ShapeDtypeStruct((n, d), jnp.float32))


@jax.jit
def kernel(x, edge_index, W, b, gamma, beta):
    n, d = x.shape
    e = edge_index.shape[1]

    npad = ((n + 1 + 127) // 128) * 128
    grain = N_TILES * CHUNK * WBODY
    nchunk = ((((e + grain - 1) // grain) * grain) // (N_TILES * CHUNK))
    pad = nchunk * CHUNK * N_TILES - e

    src = jnp.concatenate(
        [edge_index[0], jnp.zeros((pad,), edge_index.dtype)])
    dst = jnp.concatenate(
        [edge_index[1], jnp.full((pad,), n, edge_index.dtype)])
    src3 = src.reshape(N_TILES, nchunk, CHUNK)
    dst3 = dst.reshape(N_TILES, nchunk, CHUNK)

    ones_g = jnp.ones((CHUNK, GRAN), jnp.float32)
    zeros_g = jnp.zeros((npad, GRAN), jnp.float32)
    zeros_d = jnp.zeros((npad, d), jnp.float32)

    degp = _deg_call(npad, nchunk)(dst3, ones_g, zeros_g)
    h = _matmul_call(n, d)(x, W)
    hp = _scale_call(n, d, npad)(degp, h)
    mp = _msg_call(n, d, npad, nchunk)(hp, src3, dst3, zeros_d)
    return _epilogue_call(n, d, npad)(
        mp, degp, hp, x,
        b.reshape(1, d), gamma.reshape(1, d), beta.reshape(1, d))


# speedup vs baseline: 1.3676x; 1.3676x over previous
import functools

import jax
import jax.numpy as jnp
from jax import lax
from jax.experimental import pallas as pl
from jax.experimental.pallas import tpu as pltpu
from jax.experimental.pallas import tpu_sc as plsc

N_CORES = 2
N_SUBCORES = 16
N_TILES = N_CORES * N_SUBCORES
CHUNK = 128
GRAN = 128
PBUF = 2
WBODY = 8
EPS = 1e-5


def _sc_mesh():
    return plsc.VectorSubcoreMesh(core_axis_name="c", subcore_axis_name="s")


def _deg_call(npad, nchunk):
    rps = npad // N_SUBCORES

    @functools.partial(
        pl.kernel,
        out_type=jax.ShapeDtypeStruct((N_CORES, npad, GRAN), jnp.float32),
        mesh=_sc_mesh(),
        scratch_types=[
            pltpu.VMEM((nchunk, CHUNK), jnp.int32),
            pltpu.VMEM((CHUNK, GRAN), jnp.float32),
            pltpu.VMEM_SHARED((npad, GRAN), jnp.float32),
            pltpu.SemaphoreType.DMA,
        ],
    )
    def deg_kernel(dst_hbm, ones_hbm, zeros_hbm, out_hbm, idx_v, ones_v, acc,
                   sem):
        c = lax.axis_index("c")
        s = lax.axis_index("s")
        wid = c * N_SUBCORES + s
        pltpu.sync_copy(zeros_hbm.at[pl.ds(s * rps, rps)],
                        acc.at[pl.ds(s * rps, rps)])
        pltpu.sync_copy(ones_hbm, ones_v)
        pltpu.sync_copy(dst_hbm.at[wid], idx_v)
        plsc.subcore_barrier()

        @pl.loop(0, nchunk)
        def _(j):
            pltpu.async_copy(ones_v, acc.at[idx_v.at[j]], sem, add=True)

        @pl.loop(0, nchunk)
        def _(j):
            pltpu.make_async_copy(ones_hbm, ones_v, sem).wait()

        plsc.subcore_barrier()
        pltpu.sync_copy(acc.at[pl.ds(s * rps, rps)],
                        out_hbm.at[c, pl.ds(s * rps, rps)])

    return deg_kernel


def _msg_call(n, d, npad, nchunk):
    rps = npad // N_SUBCORES

    @functools.partial(
        pl.kernel,
        out_type=jax.ShapeDtypeStruct((N_CORES, npad, d), jnp.float32),
        mesh=_sc_mesh(),
        scratch_types=[
            pltpu.VMEM((nchunk, CHUNK), jnp.int32),
            pltpu.VMEM((WBODY, CHUNK), jnp.int32),
            pltpu.VMEM((PBUF, CHUNK, d), jnp.float32),
            pltpu.VMEM_SHARED((npad, d), jnp.float32),
            pltpu.SemaphoreType.DMA,
            pltpu.SemaphoreType.DMA,
            pltpu.SemaphoreType.DMA,
        ],
    )
    def msg_kernel(hp_hbm, src_hbm, dst_hbm, zeros_hbm, out_hbm,
                   src_v, dwin, bufs, acc, gsem, ssem, dsem):
        c = lax.axis_index("c")
        s = lax.axis_index("s")
        wid = c * N_SUBCORES + s
        pltpu.sync_copy(zeros_hbm.at[pl.ds(s * rps, rps)],
                        acc.at[pl.ds(s * rps, rps)])
        pltpu.sync_copy(src_hbm.at[wid], src_v)
        plsc.subcore_barrier()

        nfull = (nchunk // WBODY) * WBODY

        @pl.loop(0, nfull, step=WBODY)
        def _(j0):
            dw = pltpu.async_copy(
                dst_hbm.at[wid, pl.ds(j0, WBODY)], dwin, dsem)
            gs = [pltpu.async_copy(hp_hbm.at[src_v.at[j0]], bufs.at[0],
                                   gsem)]
            ss = []
            for b in range(WBODY):
                gs[b].wait()
                if b >= 1:
                    ss[b - 1].wait()
                if b + 1 < WBODY:
                    gs.append(pltpu.async_copy(
                        hp_hbm.at[src_v.at[j0 + b + 1]],
                        bufs.at[(b + 1) % PBUF], gsem))
                if b == 0:
                    dw.wait()
                ss.append(pltpu.async_copy(
                    bufs.at[b % PBUF], acc.at[dwin.at[b]], ssem, add=True))
            ss[WBODY - 1].wait()

        ntail = nchunk - nfull
        if ntail:
            pltpu.sync_copy(dst_hbm.at[wid, pl.ds(nfull, ntail)],
                            dwin.at[pl.ds(0, ntail)])
        for j in range(nfull, nchunk):
            b = j - nfull
            pltpu.async_copy(hp_hbm.at[src_v.at[j]],
                             bufs.at[b % PBUF], gsem).wait()
            pltpu.async_copy(bufs.at[b % PBUF], acc.at[dwin.at[b]],
                             ssem, add=True).wait()

        plsc.subcore_barrier()
        pltpu.sync_copy(acc.at[pl.ds(s * rps, rps)],
                        out_hbm.at[c, pl.ds(s * rps, rps)])

    return msg_kernel


def _matmul_call(n, d):
    def body(x_ref, w_ref, o_ref):
        o_ref[...] = lax.dot_general(
            x_ref[...], w_ref[...], (((1,), (1,)), ((), ())),
            preferred_element_type=jnp.float32)

    return pl.pallas_call(
        body, out_shape=jax.ShapeDtypeStruct((n, d), jnp.float32))


def _scale_call(n, d, npad):
    def body(dp_ref, h_ref, o_ref):
        deg = dp_ref[0, :n, 0:1] + dp_ref[1, :n, 0:1] + 1.0
        o_ref[...] = h_ref[...] * lax.rsqrt(deg)

    return pl.pallas_call(
        body, out_shape=jax.ShapeDtypeStruct((n, d), jnp.float32))


def _epilogue_call(n, d, npad):
    def body(mp_ref, dp_ref, hp_ref, x_ref, b_ref, g_ref, be_ref, o_ref):
        deg = dp_ref[0, :n, 0:1] + dp_ref[1, :n, 0:1] + 1.0
        dis = lax.rsqrt(deg)
        s = (mp_ref[0, :n, :] + mp_ref[1, :n, :] + hp_ref[...]) * dis
        s = s + b_ref[...]
        mean = jnp.mean(s, axis=0, keepdims=True)
        var = jnp.mean((s - mean) ** 2, axis=0, keepdims=True)
        y = (s - mean) * lax.rsqrt(var + EPS) * g_ref[...] + be_ref[...]
        o_ref[...] = jnp.maximum(y, 0.0) + x_ref[...]

    return pl.pallas_call(
        body, out_shape=jax.ShapeDtypeStruct((n, d), jnp.float32))


@jax.jit
def kernel(x, edge_index, W, b, gamma, beta):
    n, d = x.shape
    e = edge_index.shape[1]

    npad = ((n + 1 + 127) // 128) * 128
    grain = N_TILES * CHUNK
    nchunk = (e + grain - 1) // grain
    pad = nchunk * CHUNK * N_TILES - e

    src = jnp.concatenate(
        [edge_index[0], jnp.zeros((pad,), edge_index.dtype)])
    dst = jnp.concatenate(
        [edge_index[1], jnp.full((pad,), n, edge_index.dtype)])
    src3 = src.reshape(N_TILES, nchunk, CHUNK)
    dst3 = dst.reshape(N_TILES, nchunk, CHUNK)

    ones_g = jnp.ones((CHUNK, GRAN), jnp.float32)
    zeros_g = jnp.zeros((npad, GRAN), jnp.float32)
    zeros_d = jnp.zeros((npad, d), jnp.float32)

    degp = _deg_call(npad, nchunk)(dst3, ones_g, zeros_g)
    h = _matmul_call(n, d)(x, W)
    hp = _scale_call(n, d, npad)(degp, h)
    mp = _msg_call(n, d, npad, nchunk)(hp, src3, dst3, zeros_d)
    return _epilogue_call(n, d, npad)(
        mp, degp, hp, x,
        b.reshape(1, d), gamma.reshape(1, d), beta.reshape(1, d))
